# Initial kernel scaffold; baseline (speedup 1.0000x reference)
import jax
import jax.numpy as jnp
from jax.experimental import pallas as pl


def kernel(node_attr, edge_index, edge_attr, edge_sh, W1, b1, W2, b2, bn_weight, bn_bias):
    raise NotImplementedError("write your pallas kernel here")


# speedup vs baseline: 2.2686x; 2.2686x over previous
import functools

import jax
import jax.numpy as jnp
import numpy as np
from jax import lax
from jax.experimental import pallas as pl
from jax.experimental.pallas import tpu as pltpu
from jax.experimental.pallas import tpu_sc as plsc

IN_MUL = 16
OUT_MUL = 16
NEF = 128
WNUM = IN_MUL * OUT_MUL

NC = 2
NS = 16
NW = NC * NS
CHUNK = 128


def _gather_kernel(e_pad, n_chunks):
    per_tile = n_chunks * CHUNK
    mesh = plsc.VectorSubcoreMesh(core_axis_name="c", subcore_axis_name="s")

    @functools.partial(
        pl.kernel,
        out_type=jax.ShapeDtypeStruct((e_pad, IN_MUL), jnp.float32),
        mesh=mesh,
        compiler_params=pltpu.CompilerParams(use_tc_tiling_on_sc=False),
        scratch_types=[
            pltpu.VMEM((n_chunks, CHUNK), jnp.int32),
            pltpu.VMEM((per_tile, IN_MUL), jnp.float32),
            pltpu.SemaphoreType.DMA,
        ],
    )
    def gather_k(table_hbm, idx_hbm, out_hbm, idx_v, rows_v, sem):
        c = lax.axis_index("c")
        s = lax.axis_index("s")
        wid = s * NC + c
        pltpu.sync_copy(idx_hbm.at[wid], idx_v)
        group = 8
        for g0 in range(0, n_chunks, group):
            cps = [
                pltpu.async_copy(
                    table_hbm.at[idx_v.at[g0 + j]],
                    rows_v.at[pl.ds((g0 + j) * CHUNK, CHUNK)],
                    sem,
                )
                for j in range(min(group, n_chunks - g0))
            ]
            for cp in cps:
                cp.wait()
        pltpu.sync_copy(rows_v, out_hbm.at[pl.ds(wid * per_tile, per_tile)])

    return gather_k


def _scatter_kernel(e_pad, n_chunks, n_acc):
    per_tile = n_chunks * CHUNK
    rows_per_tile = n_acc // NS
    mesh = plsc.VectorSubcoreMesh(core_axis_name="c", subcore_axis_name="s")

    @functools.partial(
        pl.kernel,
        out_type=jax.ShapeDtypeStruct((NC, n_acc, OUT_MUL), jnp.float32),
        mesh=mesh,
        compiler_params=pltpu.CompilerParams(use_tc_tiling_on_sc=False),
        scratch_types=[
            pltpu.VMEM((n_chunks, CHUNK), jnp.int32),
            pltpu.VMEM((per_tile, OUT_MUL), jnp.float32),
            pltpu.VMEM_SHARED((n_acc, OUT_MUL), jnp.float32),
            pltpu.SemaphoreType.DMA,
        ],
    )
    def scatter_k(tp_hbm, dst_hbm, zeros_hbm, out_hbm, idx_v, rows_v, acc_sh, sem):
        c = lax.axis_index("c")
        s = lax.axis_index("s")
        wid = s * NC + c

        @pl.when(s == 0)
        def _init():
            pltpu.sync_copy(zeros_hbm, acc_sh)

        pltpu.sync_copy(dst_hbm.at[wid], idx_v)
        pltpu.sync_copy(tp_hbm.at[pl.ds(wid * per_tile, per_tile)], rows_v)
        plsc.subcore_barrier()
        for j in range(n_chunks):
            pltpu.sync_copy(
                rows_v.at[pl.ds(j * CHUNK, CHUNK)],
                acc_sh.at[idx_v.at[j]],
                add=True,
            )
        plsc.subcore_barrier()
        pltpu.sync_copy(
            acc_sh.at[pl.ds(s * rows_per_tile, rows_per_tile)],
            out_hbm.at[c, pl.ds(s * rows_per_tile, rows_per_tile)],
        )

    return scatter_k


def _tc_edge_body(ea_ref, x1_ref, sh_ref, w1_ref, b1_ref, w2_ref, b2_ref,
                  r_ref, s_ref, tp_ref):
    alpha = 1.0 / np.sqrt(IN_MUL * 1)
    h = jnp.dot(ea_ref[...], w1_ref[...], preferred_element_type=jnp.float32)
    h = jnp.maximum(h + b1_ref[...], 0.0)
    w = jnp.dot(h, w2_ref[...], preferred_element_type=jnp.float32) + b2_ref[...]
    x1s = x1_ref[...] * sh_ref[...] * alpha
    x1r = jnp.dot(x1s, r_ref[...], preferred_element_type=jnp.float32)
    tp_ref[...] = jnp.dot(x1r * w, s_ref[...], preferred_element_type=jnp.float32)


def _bn_body(p0_ref, p1_ref, na_ref, w_ref, b_ref, out_ref):
    s = p0_ref[...] + p1_ref[...] + na_ref[...]
    mean = jnp.mean(s, axis=0, keepdims=True)
    cent = s - mean
    var = jnp.mean(cent * cent, axis=0, keepdims=True)
    out_ref[...] = cent * lax.rsqrt(var + 1e-5) * w_ref[...] + b_ref[...]


def kernel(node_attr, edge_index, edge_attr, edge_sh, W1, b1, W2, b2,
           bn_weight, bn_bias):
    n = node_attr.shape[0]
    e = edge_attr.shape[0]

    per_tile_quant = NW * CHUNK
    e_pad = ((e + per_tile_quant - 1) // per_tile_quant) * per_tile_quant
    n_chunks = e_pad // (NW * CHUNK)
    pad = e_pad - e

    src = edge_index[0]
    dst = edge_index[1]
    if pad:
        src = jnp.concatenate([src, jnp.zeros((pad,), jnp.int32)])
        dst = jnp.concatenate([dst, jnp.zeros((pad,), jnp.int32)])
        edge_attr = jnp.concatenate(
            [edge_attr, jnp.zeros((pad, NEF), edge_attr.dtype)])
        edge_sh = jnp.concatenate(
            [edge_sh, jnp.zeros((pad, edge_sh.shape[1]), edge_sh.dtype)])
    src3 = src.reshape(NW, n_chunks, CHUNK)
    dst3 = dst.reshape(NW, n_chunks, CHUNK)

    x1 = _gather_kernel(e_pad, n_chunks)(node_attr, src3)

    i_idx = np.arange(WNUM) // OUT_MUL
    k_idx = np.arange(WNUM) % OUT_MUL
    R = jnp.asarray((i_idx[None, :] == np.arange(IN_MUL)[:, None])
                    .astype(np.float32))
    S = jnp.asarray((k_idx[:, None] == np.arange(OUT_MUL)[None, :])
                    .astype(np.float32))
    T = 512
    grid = (e_pad // T,)
    tp = pl.pallas_call(
        _tc_edge_body,
        grid=grid,
        in_specs=[
            pl.BlockSpec((T, NEF), lambda i: (i, 0)),
            pl.BlockSpec((T, IN_MUL), lambda i: (i, 0)),
            pl.BlockSpec((T, 1), lambda i: (i, 0)),
            pl.BlockSpec((NEF, NEF), lambda i: (0, 0)),
            pl.BlockSpec((1, NEF), lambda i: (0, 0)),
            pl.BlockSpec((NEF, WNUM), lambda i: (0, 0)),
            pl.BlockSpec((1, WNUM), lambda i: (0, 0)),
            pl.BlockSpec((IN_MUL, WNUM), lambda i: (0, 0)),
            pl.BlockSpec((WNUM, OUT_MUL), lambda i: (0, 0)),
        ],
        out_specs=pl.BlockSpec((T, OUT_MUL), lambda i: (i, 0)),
        out_shape=jax.ShapeDtypeStruct((e_pad, OUT_MUL), jnp.float32),
    )(edge_attr, x1, edge_sh, W1, b1.reshape(1, NEF), W2,
      b2.reshape(1, WNUM), R, S)

    n_acc = ((n + (NS * 8) - 1) // (NS * 8)) * (NS * 8)
    zeros = jnp.zeros((n_acc, OUT_MUL), jnp.float32)
    partials = _scatter_kernel(e_pad, n_chunks, n_acc)(tp, dst3, zeros)

    p0 = partials[0, :n]
    p1 = partials[1, :n]
    out = pl.pallas_call(
        _bn_body,
        out_shape=jax.ShapeDtypeStruct((n, OUT_MUL), jnp.float32),
    )(p0, p1, node_attr, bn_weight.reshape(1, OUT_MUL),
      bn_bias.reshape(1, OUT_MUL))
    return out


# speedup vs baseline: 3.3324x; 1.4689x over previous
import functools

import jax
import jax.numpy as jnp
import numpy as np
from jax import lax
from jax.experimental import pallas as pl
from jax.experimental.pallas import tpu as pltpu
from jax.experimental.pallas import tpu_sc as plsc

IN_MUL = 16
OUT_MUL = 16
NEF = 128
WNUM = IN_MUL * OUT_MUL

NC = 2
NS = 16
NW = NC * NS


def _gather_kernel(e, n_chunks, chunk):
    per_tile = n_chunks * chunk
    mesh = plsc.VectorSubcoreMesh(core_axis_name="c", subcore_axis_name="s")

    @functools.partial(
        pl.kernel,
        out_type=jax.ShapeDtypeStruct((e, IN_MUL), jnp.float32),
        mesh=mesh,
        compiler_params=pltpu.CompilerParams(use_tc_tiling_on_sc=False),
        scratch_types=[
            pltpu.VMEM((n_chunks, chunk), jnp.int32),
            pltpu.VMEM((per_tile, IN_MUL), jnp.float32),
            pltpu.SemaphoreType.DMA,
        ],
    )
    def gather_k(table_hbm, idx_hbm, out_hbm, idx_v, rows_v, sem):
        c = lax.axis_index("c")
        s = lax.axis_index("s")
        wid = s * NC + c
        pltpu.sync_copy(idx_hbm.at[wid], idx_v)
        group = 8
        for g0 in range(0, n_chunks, group):
            cps = [
                pltpu.async_copy(
                    table_hbm.at[idx_v.at[g0 + j]],
                    rows_v.at[pl.ds((g0 + j) * chunk, chunk)],
                    sem,
                )
                for j in range(min(group, n_chunks - g0))
            ]
            for cp in cps:
                cp.wait()
        pltpu.sync_copy(rows_v, out_hbm.at[pl.ds(wid * per_tile, per_tile)])

    return gather_k


def _scatter_kernel(e, n_chunks, chunk, n_acc):
    per_tile = n_chunks * chunk
    rows_per_tile = n_acc // NS
    mesh = plsc.VectorSubcoreMesh(core_axis_name="c", subcore_axis_name="s")

    @functools.partial(
        pl.kernel,
        out_type=jax.ShapeDtypeStruct((NC, n_acc, OUT_MUL), jnp.float32),
        mesh=mesh,
        compiler_params=pltpu.CompilerParams(use_tc_tiling_on_sc=False),
        scratch_types=[
            pltpu.VMEM((n_chunks, chunk), jnp.int32),
            pltpu.VMEM((per_tile, OUT_MUL), jnp.float32),
            pltpu.VMEM_SHARED((n_acc, OUT_MUL), jnp.float32),
            pltpu.SemaphoreType.DMA,
        ],
    )
    def scatter_k(tp_hbm, dst_hbm, zeros_hbm, out_hbm, idx_v, rows_v, acc_sh, sem):
        c = lax.axis_index("c")
        s = lax.axis_index("s")
        wid = s * NC + c

        @pl.when(s == 0)
        def _init():
            pltpu.sync_copy(zeros_hbm, acc_sh)

        pltpu.sync_copy(dst_hbm.at[wid], idx_v)
        pltpu.sync_copy(tp_hbm.at[pl.ds(wid * per_tile, per_tile)], rows_v)
        plsc.subcore_barrier()
        for j in range(n_chunks):
            pltpu.sync_copy(
                rows_v.at[pl.ds(j * chunk, chunk)],
                acc_sh.at[idx_v.at[j]],
                add=True,
            )
        plsc.subcore_barrier()
        pltpu.sync_copy(
            acc_sh.at[pl.ds(s * rows_per_tile, rows_per_tile)],
            out_hbm.at[c, pl.ds(s * rows_per_tile, rows_per_tile)],
        )

    return scatter_k


def _tc_edge_body(ea_ref, x1_ref, sh_ref, w1_ref, b1_ref, w2_ref, b2_ref,
                  r_ref, s_ref, tp_ref):
    alpha = 1.0 / np.sqrt(IN_MUL * 1)
    h = jnp.dot(ea_ref[...], w1_ref[...], preferred_element_type=jnp.float32)
    h = jnp.maximum(h + b1_ref[...], 0.0).astype(jnp.bfloat16)
    w = jnp.dot(h, w2_ref[...], preferred_element_type=jnp.float32) + b2_ref[...]
    x1s = x1_ref[...] * sh_ref[...] * alpha
    x1r = jnp.dot(x1s, r_ref[...], preferred_element_type=jnp.float32)
    tp_ref[...] = jnp.dot(x1r * w, s_ref[...], preferred_element_type=jnp.float32)


def _bn_body(p0_ref, p1_ref, na_ref, w_ref, b_ref, out_ref):
    s = p0_ref[...] + p1_ref[...] + na_ref[...]
    mean = jnp.mean(s, axis=0, keepdims=True)
    cent = s - mean
    var = jnp.mean(cent * cent, axis=0, keepdims=True)
    out_ref[...] = cent * lax.rsqrt(var + 1e-5) * w_ref[...] + b_ref[...]


def kernel(node_attr, edge_index, edge_attr, edge_sh, W1, b1, W2, b2,
           bn_weight, bn_bias):
    n = node_attr.shape[0]
    e = edge_attr.shape[0]

    chunk = 128
    while chunk > 1 and e % (NW * chunk):
        chunk -= 1
    n_chunks = e // (NW * chunk)
    assert e == NW * n_chunks * chunk

    src3 = edge_index[0].reshape(NW, n_chunks, chunk)
    dst3 = edge_index[1].reshape(NW, n_chunks, chunk)

    x1 = _gather_kernel(e, n_chunks, chunk)(node_attr, src3)

    i_idx = np.arange(WNUM) // OUT_MUL
    k_idx = np.arange(WNUM) % OUT_MUL
    R = jnp.asarray((i_idx[None, :] == np.arange(IN_MUL)[:, None])
                    .astype(np.float32))
    S = jnp.asarray((k_idx[:, None] == np.arange(OUT_MUL)[None, :])
                    .astype(np.float32))
    T = 1280
    while e % T:
        T //= 2
    grid = (e // T,)
    tp = pl.pallas_call(
        _tc_edge_body,
        grid=grid,
        in_specs=[
            pl.BlockSpec((T, NEF), lambda i: (i, 0)),
            pl.BlockSpec((T, IN_MUL), lambda i: (i, 0)),
            pl.BlockSpec((T, 1), lambda i: (i, 0)),
            pl.BlockSpec((NEF, NEF), lambda i: (0, 0)),
            pl.BlockSpec((1, NEF), lambda i: (0, 0)),
            pl.BlockSpec((NEF, WNUM), lambda i: (0, 0)),
            pl.BlockSpec((1, WNUM), lambda i: (0, 0)),
            pl.BlockSpec((IN_MUL, WNUM), lambda i: (0, 0)),
            pl.BlockSpec((WNUM, OUT_MUL), lambda i: (0, 0)),
        ],
        out_specs=pl.BlockSpec((T, OUT_MUL), lambda i: (i, 0)),
        out_shape=jax.ShapeDtypeStruct((e, OUT_MUL), jnp.float32),
    )(edge_attr.astype(jnp.bfloat16), x1, edge_sh,
      W1.astype(jnp.bfloat16), b1.reshape(1, NEF),
      W2.astype(jnp.bfloat16), b2.reshape(1, WNUM), R, S)

    n_acc = ((n + (NS * 8) - 1) // (NS * 8)) * (NS * 8)
    zeros = jnp.zeros((n_acc, OUT_MUL), jnp.float32)
    partials = _scatter_kernel(e, n_chunks, chunk, n_acc)(tp, dst3, zeros)

    p0 = partials[0, :n]
    p1 = partials[1, :n]
    out = pl.pallas_call(
        _bn_body,
        out_shape=jax.ShapeDtypeStruct((n, OUT_MUL), jnp.float32),
    )(p0, p1, node_attr, bn_weight.reshape(1, OUT_MUL),
      bn_bias.reshape(1, OUT_MUL))
    return out


# speedup vs baseline: 4.2895x; 1.2872x over previous
import functools

import jax
import jax.numpy as jnp
import numpy as np
from jax import lax
from jax.experimental import pallas as pl
from jax.experimental.pallas import tpu as pltpu
from jax.experimental.pallas import tpu_sc as plsc

IN_MUL = 16
OUT_MUL = 16
NEF = 128
WNUM = IN_MUL * OUT_MUL

NC = 2
NS = 16
NW = NC * NS
CHUNK = 128


def _tile_split(e):
    assert e % CHUNK == 0
    total_chunks = e // CHUNK
    mx = -(-total_chunks // NW)
    return total_chunks, mx


def _gather_kernel(e):
    total_chunks, mx = _tile_split(e)
    per_tile = mx * CHUNK
    last = total_chunks - (NW - 1) * mx
    assert last > 0
    mesh = plsc.VectorSubcoreMesh(core_axis_name="c", subcore_axis_name="s")

    @functools.partial(
        pl.kernel,
        out_type=jax.ShapeDtypeStruct((e, IN_MUL), jnp.float32),
        mesh=mesh,
        compiler_params=pltpu.CompilerParams(use_tc_tiling_on_sc=False),
        scratch_types=[
            pltpu.VMEM((per_tile,), jnp.int32),
            pltpu.VMEM((per_tile, IN_MUL), jnp.float32),
            pltpu.SemaphoreType.DMA,
        ],
    )
    def gather_k(table_hbm, src_hbm, out_hbm, idx_v, rows_v, sem):
        c = lax.axis_index("c")
        s = lax.axis_index("s")
        wid = s * NC + c
        base = wid * per_tile

        @pl.when(wid < NW - 1)
        def _stage_full():
            pltpu.sync_copy(src_hbm.at[pl.ds(base, per_tile)], idx_v)

        @pl.when(wid == NW - 1)
        def _stage_last():
            pltpu.sync_copy(src_hbm.at[pl.ds(base, last * CHUNK)],
                            idx_v.at[pl.ds(0, last * CHUNK)])

        group = 8
        for g0 in range(0, mx, group):
            gs = range(g0, min(g0 + group, mx))
            for g in gs:
                @pl.when(wid * mx + g < total_chunks)
                def _fire(g=g):
                    pltpu.async_copy(
                        table_hbm.at[idx_v.at[pl.ds(g * CHUNK, CHUNK)]],
                        rows_v.at[pl.ds(g * CHUNK, CHUNK)],
                        sem,
                    )
            for g in gs:
                @pl.when(wid * mx + g < total_chunks)
                def _drain(g=g):
                    pltpu.make_async_copy(
                        table_hbm.at[idx_v.at[pl.ds(g * CHUNK, CHUNK)]],
                        rows_v.at[pl.ds(g * CHUNK, CHUNK)],
                        sem,
                    ).wait()

        @pl.when(wid < NW - 1)
        def _out_full():
            pltpu.sync_copy(rows_v, out_hbm.at[pl.ds(base, per_tile)])

        @pl.when(wid == NW - 1)
        def _out_last():
            pltpu.sync_copy(rows_v.at[pl.ds(0, last * CHUNK)],
                            out_hbm.at[pl.ds(base, last * CHUNK)])

    return gather_k


def _scatter_kernel(e, n_acc):
    total_chunks, mx = _tile_split(e)
    per_tile = mx * CHUNK
    last = total_chunks - (NW - 1) * mx
    rows_per_tile = n_acc // NS
    mesh = plsc.VectorSubcoreMesh(core_axis_name="c", subcore_axis_name="s")

    @functools.partial(
        pl.kernel,
        out_type=jax.ShapeDtypeStruct((NC, n_acc, OUT_MUL), jnp.float32),
        mesh=mesh,
        compiler_params=pltpu.CompilerParams(use_tc_tiling_on_sc=False),
        scratch_types=[
            pltpu.VMEM((mx, CHUNK), jnp.int32),
            pltpu.VMEM((per_tile, OUT_MUL), jnp.float32),
            pltpu.VMEM_SHARED((n_acc, OUT_MUL), jnp.float32),
            pltpu.SemaphoreType.DMA,
            pltpu.SemaphoreType.DMA,
        ],
    )
    def scatter_k(tp_hbm, dst_hbm, zeros_hbm, out_hbm, idx_v, rows_v, acc_sh,
                  sem, sem2):
        c = lax.axis_index("c")
        s = lax.axis_index("s")
        wid = s * NC + c
        base = wid * per_tile

        @pl.when(s == 0)
        def _init():
            pltpu.sync_copy(zeros_hbm, acc_sh)

        for g in range(mx):
            @pl.when(wid * mx + g < total_chunks)
            def _idx(g=g):
                pltpu.async_copy(
                    dst_hbm.at[pl.ds(base + g * CHUNK, CHUNK)],
                    idx_v.at[g], sem2)

        @pl.when(wid < NW - 1)
        def _stage_full():
            pltpu.async_copy(tp_hbm.at[pl.ds(base, per_tile)], rows_v, sem)

        @pl.when(wid == NW - 1)
        def _stage_last():
            pltpu.async_copy(tp_hbm.at[pl.ds(base, last * CHUNK)],
                             rows_v.at[pl.ds(0, last * CHUNK)], sem)

        for g in range(mx):
            @pl.when(wid * mx + g < total_chunks)
            def _idxw(g=g):
                pltpu.make_async_copy(
                    dst_hbm.at[pl.ds(base + g * CHUNK, CHUNK)],
                    idx_v.at[g], sem2).wait()

        @pl.when(wid < NW - 1)
        def _wait_full():
            pltpu.make_async_copy(tp_hbm.at[pl.ds(base, per_tile)], rows_v,
                                  sem).wait()

        @pl.when(wid == NW - 1)
        def _wait_last():
            pltpu.make_async_copy(tp_hbm.at[pl.ds(base, last * CHUNK)],
                                  rows_v.at[pl.ds(0, last * CHUNK)],
                                  sem).wait()

        plsc.subcore_barrier()
        for g in range(mx):
            @pl.when(wid * mx + g < total_chunks)
            def _add(g=g):
                pltpu.sync_copy(
                    rows_v.at[pl.ds(g * CHUNK, CHUNK)],
                    acc_sh.at[idx_v.at[g]],
                    add=True,
                )
        plsc.subcore_barrier()
        pltpu.sync_copy(
            acc_sh.at[pl.ds(s * rows_per_tile, rows_per_tile)],
            out_hbm.at[c, pl.ds(s * rows_per_tile, rows_per_tile)],
        )

    return scatter_k


def _tc_edge_body(ea_ref, x1_ref, sh_ref, w1_ref, b1_ref, w2_ref, b2_ref,
                  r_ref, s_ref, tp_ref):
    alpha = 1.0 / np.sqrt(IN_MUL * 1)
    ea = ea_ref[...].astype(jnp.bfloat16)
    h = jnp.dot(ea, w1_ref[...], preferred_element_type=jnp.float32)
    h = jnp.maximum(h + b1_ref[...], 0.0).astype(jnp.bfloat16)
    w = jnp.dot(h, w2_ref[...], preferred_element_type=jnp.float32) + b2_ref[...]
    x1s = x1_ref[...] * sh_ref[...] * alpha
    x1r = jnp.dot(x1s, r_ref[...], preferred_element_type=jnp.float32)
    tp_ref[...] = jnp.dot(x1r * w, s_ref[...], preferred_element_type=jnp.float32)


def _bn_body(p0_ref, p1_ref, na_ref, w_ref, b_ref, out_ref):
    s = p0_ref[...] + p1_ref[...] + na_ref[...]
    mean = jnp.mean(s, axis=0, keepdims=True)
    cent = s - mean
    var = jnp.mean(cent * cent, axis=0, keepdims=True)
    out_ref[...] = cent * lax.rsqrt(var + 1e-5) * w_ref[...] + b_ref[...]


def kernel(node_attr, edge_index, edge_attr, edge_sh, W1, b1, W2, b2,
           bn_weight, bn_bias):
    n = node_attr.shape[0]
    e = edge_attr.shape[0]

    src = edge_index[0]
    dst = edge_index[1]

    x1 = _gather_kernel(e)(node_attr, src)

    i_idx = np.arange(WNUM) // OUT_MUL
    k_idx = np.arange(WNUM) % OUT_MUL
    R = jnp.asarray((i_idx[None, :] == np.arange(IN_MUL)[:, None])
                    .astype(np.float32))
    S = jnp.asarray((k_idx[:, None] == np.arange(OUT_MUL)[None, :])
                    .astype(np.float32))
    T = 4000
    while e % T:
        T //= 2
    grid = (e // T,)
    tp = pl.pallas_call(
        _tc_edge_body,
        grid=grid,
        in_specs=[
            pl.BlockSpec((T, NEF), lambda i: (i, 0)),
            pl.BlockSpec((T, IN_MUL), lambda i: (i, 0)),
            pl.BlockSpec((T, 1), lambda i: (i, 0)),
            pl.BlockSpec((NEF, NEF), lambda i: (0, 0)),
            pl.BlockSpec((1, NEF), lambda i: (0, 0)),
            pl.BlockSpec((NEF, WNUM), lambda i: (0, 0)),
            pl.BlockSpec((1, WNUM), lambda i: (0, 0)),
            pl.BlockSpec((IN_MUL, WNUM), lambda i: (0, 0)),
            pl.BlockSpec((WNUM, OUT_MUL), lambda i: (0, 0)),
        ],
        out_specs=pl.BlockSpec((T, OUT_MUL), lambda i: (i, 0)),
        out_shape=jax.ShapeDtypeStruct((e, OUT_MUL), jnp.float32),
    )(edge_attr, x1, edge_sh,
      W1.astype(jnp.bfloat16), b1.reshape(1, NEF),
      W2.astype(jnp.bfloat16), b2.reshape(1, WNUM), R, S)

    n_acc = ((n + (NS * 8) - 1) // (NS * 8)) * (NS * 8)
    zeros = jnp.zeros((n_acc, OUT_MUL), jnp.float32)
    partials = _scatter_kernel(e, n_acc)(tp, dst, zeros)

    p0 = partials[0, :n]
    p1 = partials[1, :n]
    out = pl.pallas_call(
        _bn_body,
        out_shape=jax.ShapeDtypeStruct((n, OUT_MUL), jnp.float32),
    )(p0, p1, node_attr, bn_weight.reshape(1, OUT_MUL),
      bn_bias.reshape(1, OUT_MUL))
    return out


# speedup vs baseline: 5.3624x; 1.2501x over previous
import functools

import jax
import jax.numpy as jnp
import numpy as np
from jax import lax
from jax.experimental import pallas as pl
from jax.experimental.pallas import tpu as pltpu
from jax.experimental.pallas import tpu_sc as plsc

IN_MUL = 16
OUT_MUL = 16
NEF = 128
WNUM = IN_MUL * OUT_MUL

NC = 2
NS = 16
NW = NC * NS
CHUNK = 128
GROUP = 8
RING = GROUP * CHUNK


def _tile_split(e):
    assert e % CHUNK == 0
    total_chunks = e // CHUNK
    mx = -(-total_chunks // NW)
    assert mx % GROUP == 0
    return total_chunks, mx


def _gather_kernel(e):
    total_chunks, mx = _tile_split(e)
    per_tile = mx * CHUNK
    last = total_chunks - (NW - 1) * mx
    assert last > 0
    mesh = plsc.VectorSubcoreMesh(core_axis_name="c", subcore_axis_name="s")

    @functools.partial(
        pl.kernel,
        out_type=jax.ShapeDtypeStruct((IN_MUL, e), jnp.float32),
        mesh=mesh,
        compiler_params=pltpu.CompilerParams(use_tc_tiling_on_sc=False, needs_layout_passes=False),
        scratch_types=[
            pltpu.VMEM((per_tile,), jnp.int32),
            pltpu.VMEM((RING, IN_MUL), jnp.float32),
            pltpu.VMEM((IN_MUL * per_tile,), jnp.float32),
            pltpu.SemaphoreType.DMA,
        ],
    )
    def gather_k(table_hbm, src_hbm, out_hbm, idx_v, ring_v, t_v, sem):
        c = lax.axis_index("c")
        s = lax.axis_index("s")
        wid = s * NC + c
        base = wid * per_tile

        @pl.when(wid < NW - 1)
        def _stage_full():
            pltpu.sync_copy(src_hbm.at[pl.ds(base, per_tile)], idx_v)

        @pl.when(wid == NW - 1)
        def _stage_last():
            pltpu.sync_copy(src_hbm.at[pl.ds(base, last * CHUNK)],
                            idx_v.at[pl.ds(0, last * CHUNK)])

        iota = lax.iota(jnp.int32, 16)
        for g0 in range(0, mx, GROUP):
            for g in range(g0, g0 + GROUP):
                @pl.when(wid * mx + g < total_chunks)
                def _fire(g=g):
                    pltpu.async_copy(
                        table_hbm.at[idx_v.at[pl.ds(g * CHUNK, CHUNK)]],
                        ring_v.at[pl.ds((g - g0) * CHUNK, CHUNK)],
                        sem,
                    )
            for g in range(g0, g0 + GROUP):
                @pl.when(wid * mx + g < total_chunks)
                def _drain(g=g):
                    pltpu.make_async_copy(
                        table_hbm.at[idx_v.at[pl.ds(g * CHUNK, CHUNK)]],
                        ring_v.at[pl.ds((g - g0) * CHUNK, CHUNK)],
                        sem,
                    ).wait()

            iota_pt = iota * per_tile

            def _tbody(l, g0=g0):
                v = ring_v[l]
                plsc.store_scatter(t_v, [iota_pt + (g0 * CHUNK + l)], v)

            plsc.parallel_loop(0, RING, unroll=8)(_tbody)

        for ch in range(IN_MUL):
            @pl.when(wid < NW - 1)
            def _out_full(ch=ch):
                pltpu.sync_copy(t_v.at[pl.ds(ch * per_tile, per_tile)],
                                out_hbm.at[ch, pl.ds(base, per_tile)])

            @pl.when(wid == NW - 1)
            def _out_last(ch=ch):
                pltpu.sync_copy(t_v.at[pl.ds(ch * per_tile, last * CHUNK)],
                                out_hbm.at[ch, pl.ds(base, last * CHUNK)])

    return gather_k


def _scatter_kernel(e, n_acc):
    total_chunks, mx = _tile_split(e)
    per_tile = mx * CHUNK
    last = total_chunks - (NW - 1) * mx
    rows_per_tile = n_acc // NS
    mesh = plsc.VectorSubcoreMesh(core_axis_name="c", subcore_axis_name="s")

    @functools.partial(
        pl.kernel,
        out_type=jax.ShapeDtypeStruct((NC, n_acc, OUT_MUL), jnp.float32),
        mesh=mesh,
        compiler_params=pltpu.CompilerParams(use_tc_tiling_on_sc=False, needs_layout_passes=False),
        scratch_types=[
            pltpu.VMEM((mx, CHUNK), jnp.int32),
            pltpu.VMEM((RING, OUT_MUL), jnp.float32),
            pltpu.VMEM((OUT_MUL * per_tile,), jnp.float32),
            pltpu.VMEM_SHARED((n_acc, OUT_MUL), jnp.float32),
            pltpu.SemaphoreType.DMA,
            pltpu.SemaphoreType.DMA,
        ],
    )
    def scatter_k(tp_hbm, dst_hbm, zeros_hbm, out_hbm, idx_v, ring_v, t_v,
                  acc_sh, sem, sem2):
        c = lax.axis_index("c")
        s = lax.axis_index("s")
        wid = s * NC + c
        base = wid * per_tile

        @pl.when(s == 0)
        def _init():
            pltpu.sync_copy(zeros_hbm, acc_sh)

        for g in range(mx):
            @pl.when(wid * mx + g < total_chunks)
            def _idx(g=g):
                pltpu.async_copy(
                    dst_hbm.at[pl.ds(base + g * CHUNK, CHUNK)],
                    idx_v.at[g], sem2)

        for ch in range(OUT_MUL):
            @pl.when(wid < NW - 1)
            def _stage_full(ch=ch):
                pltpu.async_copy(tp_hbm.at[ch, pl.ds(base, per_tile)],
                                 t_v.at[pl.ds(ch * per_tile, per_tile)], sem)

            @pl.when(wid == NW - 1)
            def _stage_last(ch=ch):
                pltpu.async_copy(tp_hbm.at[ch, pl.ds(base, last * CHUNK)],
                                 t_v.at[pl.ds(ch * per_tile, last * CHUNK)],
                                 sem)

        for g in range(mx):
            @pl.when(wid * mx + g < total_chunks)
            def _idxw(g=g):
                pltpu.make_async_copy(
                    dst_hbm.at[pl.ds(base + g * CHUNK, CHUNK)],
                    idx_v.at[g], sem2).wait()

        for ch in range(OUT_MUL):
            @pl.when(wid < NW - 1)
            def _wait_full(ch=ch):
                pltpu.make_async_copy(
                    tp_hbm.at[ch, pl.ds(base, per_tile)],
                    t_v.at[pl.ds(ch * per_tile, per_tile)], sem).wait()

            @pl.when(wid == NW - 1)
            def _wait_last(ch=ch):
                pltpu.make_async_copy(
                    tp_hbm.at[ch, pl.ds(base, last * CHUNK)],
                    t_v.at[pl.ds(ch * per_tile, last * CHUNK)], sem).wait()

        plsc.subcore_barrier()
        iota = lax.iota(jnp.int32, 16)
        iota_pt = iota * per_tile
        for g0 in range(0, mx, GROUP):
            def _tbody(l, g0=g0):
                v = plsc.load_gather(t_v, [iota_pt + (g0 * CHUNK + l)])
                ring_v[l] = v

            plsc.parallel_loop(0, RING, unroll=8)(_tbody)

            for g in range(g0, g0 + GROUP):
                @pl.when(wid * mx + g < total_chunks)
                def _add(g=g):
                    pltpu.sync_copy(
                        ring_v.at[pl.ds((g - g0) * CHUNK, CHUNK)],
                        acc_sh.at[idx_v.at[g]],
                        add=True,
                    )
        plsc.subcore_barrier()
        pltpu.sync_copy(
            acc_sh.at[pl.ds(s * rows_per_tile, rows_per_tile)],
            out_hbm.at[c, pl.ds(s * rows_per_tile, rows_per_tile)],
        )

    return scatter_k


def _tc_edge_body(ea_ref, x1t_ref, sh_ref, w1_ref, b1_ref, w2_ref, b2_ref,
                  r_ref, s_ref, tpt_ref):
    ea = ea_ref[...].astype(jnp.bfloat16)
    h = jnp.dot(ea, w1_ref[...], preferred_element_type=jnp.float32)
    h = jnp.maximum(h + b1_ref[...], 0.0).astype(jnp.bfloat16)
    w = jnp.dot(h, w2_ref[...], preferred_element_type=jnp.float32) + b2_ref[...]
    x1r = lax.dot_general(x1t_ref[...], r_ref[...],
                          (((0,), (0,)), ((), ())),
                          preferred_element_type=jnp.float32)
    prod = x1r * w
    tpt = lax.dot_general(s_ref[...], prod,
                          (((0,), (1,)), ((), ())),
                          preferred_element_type=jnp.float32)
    tpt_ref[...] = tpt * sh_ref[...]


def _bn_body(p0_ref, p1_ref, na_ref, w_ref, b_ref, out_ref):
    s = p0_ref[...] + p1_ref[...] + na_ref[...]
    mean = jnp.mean(s, axis=0, keepdims=True)
    cent = s - mean
    var = jnp.mean(cent * cent, axis=0, keepdims=True)
    out_ref[...] = cent * lax.rsqrt(var + 1e-5) * w_ref[...] + b_ref[...]


def kernel(node_attr, edge_index, edge_attr, edge_sh, W1, b1, W2, b2,
           bn_weight, bn_bias):
    n = node_attr.shape[0]
    e = edge_attr.shape[0]

    src = edge_index[0]
    dst = edge_index[1]

    x1t = _gather_kernel(e)(node_attr, src)

    alpha = 1.0 / np.sqrt(IN_MUL * 1)
    i_idx = np.arange(WNUM) // OUT_MUL
    k_idx = np.arange(WNUM) % OUT_MUL
    R = jnp.asarray((i_idx[None, :] == np.arange(IN_MUL)[:, None])
                    .astype(np.float32) * alpha)
    S = jnp.asarray((k_idx[:, None] == np.arange(OUT_MUL)[None, :])
                    .astype(np.float32))
    T = 3200
    while e % T:
        T //= 2
    grid = (e // T,)
    sh_row = edge_sh.reshape(1, e)
    tpt = pl.pallas_call(
        _tc_edge_body,
        grid=grid,
        in_specs=[
            pl.BlockSpec((T, NEF), lambda i: (i, 0)),
            pl.BlockSpec((IN_MUL, T), lambda i: (0, i)),
            pl.BlockSpec((1, T), lambda i: (0, i)),
            pl.BlockSpec((NEF, NEF), lambda i: (0, 0)),
            pl.BlockSpec((1, NEF), lambda i: (0, 0)),
            pl.BlockSpec((NEF, WNUM), lambda i: (0, 0)),
            pl.BlockSpec((1, WNUM), lambda i: (0, 0)),
            pl.BlockSpec((IN_MUL, WNUM), lambda i: (0, 0)),
            pl.BlockSpec((WNUM, OUT_MUL), lambda i: (0, 0)),
        ],
        out_specs=pl.BlockSpec((OUT_MUL, T), lambda i: (0, i)),
        out_shape=jax.ShapeDtypeStruct((OUT_MUL, e), jnp.float32),
    )(edge_attr, x1t, sh_row,
      W1.astype(jnp.bfloat16), b1.reshape(1, NEF),
      W2.astype(jnp.bfloat16), b2.reshape(1, WNUM), R, S)

    n_acc = ((n + (NS * 8) - 1) // (NS * 8)) * (NS * 8)
    zeros = jnp.zeros((n_acc, OUT_MUL), jnp.float32)
    partials = _scatter_kernel(e, n_acc)(tpt, dst, zeros)

    p0 = partials[0, :n]
    p1 = partials[1, :n]
    out = pl.pallas_call(
        _bn_body,
        out_shape=jax.ShapeDtypeStruct((n, OUT_MUL), jnp.float32),
    )(p0, p1, node_attr, bn_weight.reshape(1, OUT_MUL),
      bn_bias.reshape(1, OUT_MUL))
    return out


# speedup vs baseline: 5.6086x; 1.0459x over previous
import functools

import jax
import jax.numpy as jnp
import numpy as np
from jax import lax
from jax.experimental import pallas as pl
from jax.experimental.pallas import tpu as pltpu
from jax.experimental.pallas import tpu_sc as plsc

IN_MUL = 16
OUT_MUL = 16
NEF = 128
WNUM = IN_MUL * OUT_MUL

NC = 2
NS = 16
NW = NC * NS
CHUNK = 128
GROUP = 8
RING = GROUP * CHUNK


def _tile_split(e):
    assert e % CHUNK == 0
    total_chunks = e // CHUNK
    mx = -(-total_chunks // NW)
    assert mx % GROUP == 0
    return total_chunks, mx


def _gather_kernel(e):
    total_chunks, mx = _tile_split(e)
    per_tile = mx * CHUNK
    last = total_chunks - (NW - 1) * mx
    assert last > 0
    mesh = plsc.VectorSubcoreMesh(core_axis_name="c", subcore_axis_name="s")

    @functools.partial(
        pl.kernel,
        out_type=jax.ShapeDtypeStruct((IN_MUL, e), jnp.float32),
        mesh=mesh,
        compiler_params=pltpu.CompilerParams(use_tc_tiling_on_sc=False, needs_layout_passes=False),
        scratch_types=[
            pltpu.VMEM((per_tile,), jnp.int32),
            pltpu.VMEM((2 * RING, IN_MUL), jnp.float32),
            pltpu.VMEM((IN_MUL * per_tile,), jnp.float32),
            pltpu.SemaphoreType.DMA,
        ],
    )
    def gather_k(table_hbm, src_hbm, out_hbm, idx_v, ring_v, t_v, sem):
        c = lax.axis_index("c")
        s = lax.axis_index("s")
        wid = s * NC + c
        base = wid * per_tile

        @pl.when(wid < NW - 1)
        def _stage_full():
            pltpu.sync_copy(src_hbm.at[pl.ds(base, per_tile)], idx_v)

        @pl.when(wid == NW - 1)
        def _stage_last():
            pltpu.sync_copy(src_hbm.at[pl.ds(base, last * CHUNK)],
                            idx_v.at[pl.ds(0, last * CHUNK)])

        iota = lax.iota(jnp.int32, 16)
        iota_pt = iota * per_tile
        n_groups = mx // GROUP

        def _fire(g0, half):
            for g in range(g0, g0 + GROUP):
                @pl.when(wid * mx + g < total_chunks)
                def _f(g=g):
                    pltpu.async_copy(
                        table_hbm.at[idx_v.at[pl.ds(g * CHUNK, CHUNK)]],
                        ring_v.at[pl.ds(half * RING + (g - g0) * CHUNK,
                                        CHUNK)],
                        sem,
                    )

        def _drain(g0, half):
            for g in range(g0, g0 + GROUP):
                @pl.when(wid * mx + g < total_chunks)
                def _d(g=g):
                    pltpu.make_async_copy(
                        table_hbm.at[idx_v.at[pl.ds(g * CHUNK, CHUNK)]],
                        ring_v.at[pl.ds(half * RING + (g - g0) * CHUNK,
                                        CHUNK)],
                        sem,
                    ).wait()

        _fire(0, 0)
        for gi in range(n_groups):
            g0 = gi * GROUP
            half = gi % 2
            _drain(g0, half)
            if gi + 1 < n_groups:
                _fire(g0 + GROUP, 1 - half)

            def _tbody(l, g0=g0, half=half):
                v = ring_v[half * RING + l]
                plsc.store_scatter(t_v, [iota_pt + (g0 * CHUNK + l)], v)

            plsc.parallel_loop(0, RING, unroll=8)(_tbody)

        for ch in range(IN_MUL):
            @pl.when(wid < NW - 1)
            def _out_full(ch=ch):
                pltpu.sync_copy(t_v.at[pl.ds(ch * per_tile, per_tile)],
                                out_hbm.at[ch, pl.ds(base, per_tile)])

            @pl.when(wid == NW - 1)
            def _out_last(ch=ch):
                pltpu.sync_copy(t_v.at[pl.ds(ch * per_tile, last * CHUNK)],
                                out_hbm.at[ch, pl.ds(base, last * CHUNK)])

    return gather_k


def _scatter_kernel(e, n_acc):
    total_chunks, mx = _tile_split(e)
    per_tile = mx * CHUNK
    last = total_chunks - (NW - 1) * mx
    rows_per_tile = n_acc // NS
    mesh = plsc.VectorSubcoreMesh(core_axis_name="c", subcore_axis_name="s")

    @functools.partial(
        pl.kernel,
        out_type=jax.ShapeDtypeStruct((NC, n_acc, OUT_MUL), jnp.float32),
        mesh=mesh,
        compiler_params=pltpu.CompilerParams(use_tc_tiling_on_sc=False, needs_layout_passes=False),
        scratch_types=[
            pltpu.VMEM((mx, CHUNK), jnp.int32),
            pltpu.VMEM((2 * RING, OUT_MUL), jnp.float32),
            pltpu.VMEM((OUT_MUL * per_tile,), jnp.float32),
            pltpu.VMEM_SHARED((n_acc, OUT_MUL), jnp.float32),
            pltpu.SemaphoreType.DMA,
            pltpu.SemaphoreType.DMA,
        ],
    )
    def scatter_k(tp_hbm, dst_hbm, zeros_hbm, out_hbm, idx_v, ring_v, t_v,
                  acc_sh, sem, sem2):
        c = lax.axis_index("c")
        s = lax.axis_index("s")
        wid = s * NC + c
        base = wid * per_tile

        @pl.when(s == 0)
        def _init():
            pltpu.sync_copy(zeros_hbm, acc_sh)

        for g in range(mx):
            @pl.when(wid * mx + g < total_chunks)
            def _idx(g=g):
                pltpu.async_copy(
                    dst_hbm.at[pl.ds(base + g * CHUNK, CHUNK)],
                    idx_v.at[g], sem2)

        for ch in range(OUT_MUL):
            @pl.when(wid < NW - 1)
            def _stage_full(ch=ch):
                pltpu.async_copy(tp_hbm.at[ch, pl.ds(base, per_tile)],
                                 t_v.at[pl.ds(ch * per_tile, per_tile)], sem)

            @pl.when(wid == NW - 1)
            def _stage_last(ch=ch):
                pltpu.async_copy(tp_hbm.at[ch, pl.ds(base, last * CHUNK)],
                                 t_v.at[pl.ds(ch * per_tile, last * CHUNK)],
                                 sem)

        for g in range(mx):
            @pl.when(wid * mx + g < total_chunks)
            def _idxw(g=g):
                pltpu.make_async_copy(
                    dst_hbm.at[pl.ds(base + g * CHUNK, CHUNK)],
                    idx_v.at[g], sem2).wait()

        for ch in range(OUT_MUL):
            @pl.when(wid < NW - 1)
            def _wait_full(ch=ch):
                pltpu.make_async_copy(
                    tp_hbm.at[ch, pl.ds(base, per_tile)],
                    t_v.at[pl.ds(ch * per_tile, per_tile)], sem).wait()

            @pl.when(wid == NW - 1)
            def _wait_last(ch=ch):
                pltpu.make_async_copy(
                    tp_hbm.at[ch, pl.ds(base, last * CHUNK)],
                    t_v.at[pl.ds(ch * per_tile, last * CHUNK)], sem).wait()

        plsc.subcore_barrier()
        iota = lax.iota(jnp.int32, 16)
        iota_pt = iota * per_tile
        n_groups = mx // GROUP

        def _transpose(g0, half):
            def _tbody(l, g0=g0, half=half):
                v = plsc.load_gather(t_v, [iota_pt + (g0 * CHUNK + l)])
                ring_v[half * RING + l] = v

            plsc.parallel_loop(0, RING, unroll=8)(_tbody)

        def _fire_adds(g0, half):
            for g in range(g0, g0 + GROUP):
                @pl.when(wid * mx + g < total_chunks)
                def _a(g=g):
                    pltpu.async_copy(
                        ring_v.at[pl.ds(half * RING + (g - g0) * CHUNK,
                                        CHUNK)],
                        acc_sh.at[idx_v.at[g]],
                        sem, add=True,
                    )

        def _drain_adds(g0, half):
            for g in range(g0, g0 + GROUP):
                @pl.when(wid * mx + g < total_chunks)
                def _w(g=g):
                    pltpu.make_async_copy(
                        ring_v.at[pl.ds(half * RING + (g - g0) * CHUNK,
                                        CHUNK)],
                        acc_sh.at[idx_v.at[g]],
                        sem,
                    ).wait()

        _transpose(0, 0)
        for gi in range(n_groups):
            g0 = gi * GROUP
            half = gi % 2
            _fire_adds(g0, half)
            if gi + 1 < n_groups:
                _transpose(g0 + GROUP, 1 - half)
            _drain_adds(g0, half)
        plsc.subcore_barrier()
        pltpu.sync_copy(
            acc_sh.at[pl.ds(s * rows_per_tile, rows_per_tile)],
            out_hbm.at[c, pl.ds(s * rows_per_tile, rows_per_tile)],
        )

    return scatter_k


def _tc_edge_body(ea_ref, x1t_ref, sh_ref, w1_ref, b1_ref, w2_ref, b2_ref,
                  r_ref, s_ref, tpt_ref):
    ea = ea_ref[...].astype(jnp.bfloat16)
    h = jnp.dot(ea, w1_ref[...], preferred_element_type=jnp.float32)
    h = jnp.maximum(h + b1_ref[...], 0.0).astype(jnp.bfloat16)
    w = jnp.dot(h, w2_ref[...], preferred_element_type=jnp.float32) + b2_ref[...]
    x1r = lax.dot_general(x1t_ref[...].astype(jnp.bfloat16), r_ref[...],
                          (((0,), (0,)), ((), ())),
                          preferred_element_type=jnp.float32)
    prod = (x1r * w).astype(jnp.bfloat16)
    tpt = lax.dot_general(s_ref[...], prod,
                          (((0,), (1,)), ((), ())),
                          preferred_element_type=jnp.float32)
    tpt_ref[...] = tpt * sh_ref[...]


def _bn_body(p0_ref, p1_ref, na_ref, w_ref, b_ref, out_ref):
    s = p0_ref[...] + p1_ref[...] + na_ref[...]
    mean = jnp.mean(s, axis=0, keepdims=True)
    cent = s - mean
    var = jnp.mean(cent * cent, axis=0, keepdims=True)
    out_ref[...] = cent * lax.rsqrt(var + 1e-5) * w_ref[...] + b_ref[...]


def kernel(node_attr, edge_index, edge_attr, edge_sh, W1, b1, W2, b2,
           bn_weight, bn_bias):
    n = node_attr.shape[0]
    e = edge_attr.shape[0]

    src = edge_index[0]
    dst = edge_index[1]

    x1t = _gather_kernel(e)(node_attr, src)

    alpha = 1.0 / np.sqrt(IN_MUL * 1)
    i_idx = np.arange(WNUM) // OUT_MUL
    k_idx = np.arange(WNUM) % OUT_MUL
    R = jnp.asarray((i_idx[None, :] == np.arange(IN_MUL)[:, None])
                    .astype(np.float32) * alpha).astype(jnp.bfloat16)
    S = jnp.asarray((k_idx[:, None] == np.arange(OUT_MUL)[None, :])
                    .astype(np.float32)).astype(jnp.bfloat16)
    T = 3200
    while e % T:
        T //= 2
    grid = (e // T,)
    sh_row = edge_sh.reshape(1, e)
    tpt = pl.pallas_call(
        _tc_edge_body,
        grid=grid,
        in_specs=[
            pl.BlockSpec((T, NEF), lambda i: (i, 0)),
            pl.BlockSpec((IN_MUL, T), lambda i: (0, i)),
            pl.BlockSpec((1, T), lambda i: (0, i)),
            pl.BlockSpec((NEF, NEF), lambda i: (0, 0)),
            pl.BlockSpec((1, NEF), lambda i: (0, 0)),
            pl.BlockSpec((NEF, WNUM), lambda i: (0, 0)),
            pl.BlockSpec((1, WNUM), lambda i: (0, 0)),
            pl.BlockSpec((IN_MUL, WNUM), lambda i: (0, 0)),
            pl.BlockSpec((WNUM, OUT_MUL), lambda i: (0, 0)),
        ],
        out_specs=pl.BlockSpec((OUT_MUL, T), lambda i: (0, i)),
        out_shape=jax.ShapeDtypeStruct((OUT_MUL, e), jnp.float32),
    )(edge_attr, x1t, sh_row,
      W1.astype(jnp.bfloat16), b1.reshape(1, NEF),
      W2.astype(jnp.bfloat16), b2.reshape(1, WNUM), R, S)

    n_acc = ((n + (NS * 8) - 1) // (NS * 8)) * (NS * 8)
    zeros = jnp.zeros((n_acc, OUT_MUL), jnp.float32)
    partials = _scatter_kernel(e, n_acc)(tpt, dst, zeros)

    p0 = partials[0, :n]
    p1 = partials[1, :n]
    out = pl.pallas_call(
        _bn_body,
        out_shape=jax.ShapeDtypeStruct((n, OUT_MUL), jnp.float32),
    )(p0, p1, node_attr, bn_weight.reshape(1, OUT_MUL),
      bn_bias.reshape(1, OUT_MUL))
    return out


# speedup vs baseline: 6.0155x; 1.0726x over previous
import functools

import jax
import jax.numpy as jnp
import numpy as np
from jax import lax
from jax.experimental import pallas as pl
from jax.experimental.pallas import tpu as pltpu
from jax.experimental.pallas import tpu_sc as plsc

IN_MUL = 16
OUT_MUL = 16
NEF = 128
WNUM = IN_MUL * OUT_MUL

NC = 2
NS = 16
NW = NC * NS
CHUNK = 128
GROUP = 8
RING = GROUP * CHUNK


def _tile_split(e):
    assert e % CHUNK == 0
    total_chunks = e // CHUNK
    mx = -(-total_chunks // NW)
    assert mx % GROUP == 0
    return total_chunks, mx


def _gather_kernel(e):
    total_chunks, mx = _tile_split(e)
    per_tile = mx * CHUNK
    last = total_chunks - (NW - 1) * mx
    assert last > 0
    mesh = plsc.VectorSubcoreMesh(core_axis_name="c", subcore_axis_name="s")

    @functools.partial(
        pl.kernel,
        out_type=jax.ShapeDtypeStruct((e * IN_MUL,), jnp.float32),
        mesh=mesh,
        compiler_params=pltpu.CompilerParams(use_tc_tiling_on_sc=False, needs_layout_passes=False),
        scratch_types=[
            pltpu.VMEM((per_tile,), jnp.int32),
            pltpu.VMEM((2 * RING, IN_MUL), jnp.float32),
            pltpu.VMEM((IN_MUL * per_tile,), jnp.float32),
            pltpu.SemaphoreType.DMA,
        ],
    )
    def gather_k(table_hbm, src_hbm, out_hbm, idx_v, ring_v, t_v, sem):
        c = lax.axis_index("c")
        s = lax.axis_index("s")
        wid = s * NC + c
        base = wid * per_tile

        @pl.when(wid < NW - 1)
        def _stage_full():
            pltpu.sync_copy(src_hbm.at[pl.ds(base, per_tile)], idx_v)

        @pl.when(wid == NW - 1)
        def _stage_last():
            pltpu.sync_copy(src_hbm.at[pl.ds(base, last * CHUNK)],
                            idx_v.at[pl.ds(0, last * CHUNK)])

        iota = lax.iota(jnp.int32, 16)
        iota_128 = iota * 128
        n_groups = mx // GROUP

        def _fire(g0, half):
            for g in range(g0, g0 + GROUP):
                @pl.when(wid * mx + g < total_chunks)
                def _f(g=g):
                    pltpu.async_copy(
                        table_hbm.at[idx_v.at[pl.ds(g * CHUNK, CHUNK)]],
                        ring_v.at[pl.ds(half * RING + (g - g0) * CHUNK,
                                        CHUNK)],
                        sem,
                    )

        def _drain(g0, half):
            for g in range(g0, g0 + GROUP):
                @pl.when(wid * mx + g < total_chunks)
                def _d(g=g):
                    pltpu.make_async_copy(
                        table_hbm.at[idx_v.at[pl.ds(g * CHUNK, CHUNK)]],
                        ring_v.at[pl.ds(half * RING + (g - g0) * CHUNK,
                                        CHUNK)],
                        sem,
                    ).wait()

        _fire(0, 0)
        for gi in range(n_groups):
            g0 = gi * GROUP
            half = gi % 2
            _drain(g0, half)
            if gi + 1 < n_groups:
                _fire(g0 + GROUP, 1 - half)

            def _tbody(l, g0=g0, half=half):
                v = ring_v[half * RING + l]
                le = g0 * CHUNK + l
                off = (le >> 7) * (16 * 128) + (le & 127)
                plsc.store_scatter(t_v, [iota_128 + off], v)

            plsc.parallel_loop(0, RING, unroll=8)(_tbody)

        @pl.when(wid < NW - 1)
        def _out_full():
            pltpu.sync_copy(t_v,
                            out_hbm.at[pl.ds(base * IN_MUL,
                                             per_tile * IN_MUL)])

        @pl.when(wid == NW - 1)
        def _out_last():
            pltpu.sync_copy(t_v.at[pl.ds(0, last * CHUNK * IN_MUL)],
                            out_hbm.at[pl.ds(base * IN_MUL,
                                             last * CHUNK * IN_MUL)])

    return gather_k


def _scatter_kernel(e, n_acc):
    total_chunks, mx = _tile_split(e)
    per_tile = mx * CHUNK
    last = total_chunks - (NW - 1) * mx
    rows_per_tile = n_acc // NS
    mesh = plsc.VectorSubcoreMesh(core_axis_name="c", subcore_axis_name="s")

    @functools.partial(
        pl.kernel,
        out_type=jax.ShapeDtypeStruct((NC, n_acc, OUT_MUL), jnp.float32),
        mesh=mesh,
        compiler_params=pltpu.CompilerParams(use_tc_tiling_on_sc=False, needs_layout_passes=False),
        scratch_types=[
            pltpu.VMEM((mx, CHUNK), jnp.int32),
            pltpu.VMEM((2 * RING, OUT_MUL), jnp.float32),
            pltpu.VMEM((OUT_MUL * per_tile,), jnp.float32),
            pltpu.VMEM_SHARED((n_acc, OUT_MUL), jnp.float32),
            pltpu.SemaphoreType.DMA,
            pltpu.SemaphoreType.DMA,
        ],
    )
    def scatter_k(tp_hbm, dst_hbm, zeros_hbm, out_hbm, idx_v, ring_v, t_v,
                  acc_sh, sem, sem2):
        c = lax.axis_index("c")
        s = lax.axis_index("s")
        wid = s * NC + c
        base = wid * per_tile

        @pl.when(s == 0)
        def _init():
            pltpu.sync_copy(zeros_hbm, acc_sh)

        for g in range(mx):
            @pl.when(wid * mx + g < total_chunks)
            def _idx(g=g):
                pltpu.async_copy(
                    dst_hbm.at[pl.ds(base + g * CHUNK, CHUNK)],
                    idx_v.at[g], sem2)

        @pl.when(wid < NW - 1)
        def _stage_full():
            pltpu.async_copy(
                tp_hbm.at[pl.ds(base * OUT_MUL, per_tile * OUT_MUL)],
                t_v, sem)

        @pl.when(wid == NW - 1)
        def _stage_last():
            pltpu.async_copy(
                tp_hbm.at[pl.ds(base * OUT_MUL, last * CHUNK * OUT_MUL)],
                t_v.at[pl.ds(0, last * CHUNK * OUT_MUL)], sem)

        for g in range(mx):
            @pl.when(wid * mx + g < total_chunks)
            def _idxw(g=g):
                pltpu.make_async_copy(
                    dst_hbm.at[pl.ds(base + g * CHUNK, CHUNK)],
                    idx_v.at[g], sem2).wait()

        @pl.when(wid < NW - 1)
        def _wait_full():
            pltpu.make_async_copy(
                tp_hbm.at[pl.ds(base * OUT_MUL, per_tile * OUT_MUL)],
                t_v, sem).wait()

        @pl.when(wid == NW - 1)
        def _wait_last():
            pltpu.make_async_copy(
                tp_hbm.at[pl.ds(base * OUT_MUL, last * CHUNK * OUT_MUL)],
                t_v.at[pl.ds(0, last * CHUNK * OUT_MUL)], sem).wait()

        plsc.subcore_barrier()
        iota = lax.iota(jnp.int32, 16)
        iota_128 = iota * 128
        n_groups = mx // GROUP

        def _transpose(g0, half):
            def _tbody(l, g0=g0, half=half):
                le = g0 * CHUNK + l
                off = (le >> 7) * (16 * 128) + (le & 127)
                v = plsc.load_gather(t_v, [iota_128 + off])
                ring_v[half * RING + l] = v

            plsc.parallel_loop(0, RING, unroll=8)(_tbody)

        def _fire_adds(g0, half):
            for g in range(g0, g0 + GROUP):
                @pl.when(wid * mx + g < total_chunks)
                def _a(g=g):
                    pltpu.async_copy(
                        ring_v.at[pl.ds(half * RING + (g - g0) * CHUNK,
                                        CHUNK)],
                        acc_sh.at[idx_v.at[g]],
                        sem, add=True,
                    )

        def _drain_adds(g0, half):
            for g in range(g0, g0 + GROUP):
                @pl.when(wid * mx + g < total_chunks)
                def _w(g=g):
                    pltpu.make_async_copy(
                        ring_v.at[pl.ds(half * RING + (g - g0) * CHUNK,
                                        CHUNK)],
                        acc_sh.at[idx_v.at[g]],
                        sem,
                    ).wait()

        _transpose(0, 0)
        for gi in range(n_groups):
            g0 = gi * GROUP
            half = gi % 2
            _fire_adds(g0, half)
            if gi + 1 < n_groups:
                _transpose(g0 + GROUP, 1 - half)
            _drain_adds(g0, half)
        plsc.subcore_barrier()
        pltpu.sync_copy(
            acc_sh.at[pl.ds(s * rows_per_tile, rows_per_tile)],
            out_hbm.at[c, pl.ds(s * rows_per_tile, rows_per_tile)],
        )

    return scatter_k


def _tc_edge_body(ea_ref, x1t_ref, sh_ref, w1_ref, b1_ref, w2_ref, b2_ref,
                  r_ref, s_ref, tpt_ref):
    t = ea_ref.shape[0]
    ea = ea_ref[...].astype(jnp.bfloat16)
    h = jnp.dot(ea, w1_ref[...], preferred_element_type=jnp.float32)
    h = jnp.maximum(h + b1_ref[...], 0.0).astype(jnp.bfloat16)
    w = jnp.dot(h, w2_ref[...], preferred_element_type=jnp.float32) + b2_ref[...]
    tb = t // 128
    x1e = jnp.swapaxes(x1t_ref[...], 1, 2).reshape(t, IN_MUL)
    x1r = jnp.dot(x1e.astype(jnp.bfloat16), r_ref[...],
                  preferred_element_type=jnp.float32)
    prod = (x1r * w).astype(jnp.bfloat16)
    tp = jnp.dot(prod, s_ref[...], preferred_element_type=jnp.float32)
    tp3 = jnp.swapaxes(tp.reshape(tb, 128, OUT_MUL), 1, 2)
    tpt_ref[...] = tp3 * sh_ref[...].reshape(tb, 1, 128)


def _bn_body(p0_ref, p1_ref, na_ref, w_ref, b_ref, out_ref):
    s = p0_ref[...] + p1_ref[...] + na_ref[...]
    mean = jnp.mean(s, axis=0, keepdims=True)
    cent = s - mean
    var = jnp.mean(cent * cent, axis=0, keepdims=True)
    out_ref[...] = cent * lax.rsqrt(var + 1e-5) * w_ref[...] + b_ref[...]


def kernel(node_attr, edge_index, edge_attr, edge_sh, W1, b1, W2, b2,
           bn_weight, bn_bias):
    n = node_attr.shape[0]
    e = edge_attr.shape[0]

    src = edge_index[0]
    dst = edge_index[1]

    x1t = _gather_kernel(e)(node_attr, src)

    alpha = 1.0 / np.sqrt(IN_MUL * 1)
    i_idx = np.arange(WNUM) // OUT_MUL
    k_idx = np.arange(WNUM) % OUT_MUL
    R = jnp.asarray((i_idx[None, :] == np.arange(IN_MUL)[:, None])
                    .astype(np.float32) * alpha).astype(jnp.bfloat16)
    S = jnp.asarray((k_idx[:, None] == np.arange(OUT_MUL)[None, :])
                    .astype(np.float32)).astype(jnp.bfloat16)
    T = 6400
    while e % T:
        T //= 2
    grid = (e // T,)
    ec = e // 128
    tc = T // 128
    sh3 = edge_sh.reshape(ec, 1, 128)
    x1t3 = x1t.reshape(ec, IN_MUL, 128)
    tpt3 = pl.pallas_call(
        _tc_edge_body,
        grid=grid,
        in_specs=[
            pl.BlockSpec((T, NEF), lambda i: (i, 0)),
            pl.BlockSpec((tc, IN_MUL, 128), lambda i: (i, 0, 0)),
            pl.BlockSpec((tc, 1, 128), lambda i: (i, 0, 0)),
            pl.BlockSpec((NEF, NEF), lambda i: (0, 0)),
            pl.BlockSpec((1, NEF), lambda i: (0, 0)),
            pl.BlockSpec((NEF, WNUM), lambda i: (0, 0)),
            pl.BlockSpec((1, WNUM), lambda i: (0, 0)),
            pl.BlockSpec((IN_MUL, WNUM), lambda i: (0, 0)),
            pl.BlockSpec((WNUM, OUT_MUL), lambda i: (0, 0)),
        ],
        out_specs=pl.BlockSpec((tc, OUT_MUL, 128), lambda i: (i, 0, 0)),
        out_shape=jax.ShapeDtypeStruct((ec, OUT_MUL, 128), jnp.float32),
    )(edge_attr, x1t3, sh3,
      W1.astype(jnp.bfloat16), b1.reshape(1, NEF),
      W2.astype(jnp.bfloat16), b2.reshape(1, WNUM), R, S)
    tpt = tpt3.reshape(e * OUT_MUL)

    n_acc = ((n + (NS * 8) - 1) // (NS * 8)) * (NS * 8)
    zeros = jnp.zeros((n_acc, OUT_MUL), jnp.float32)
    partials = _scatter_kernel(e, n_acc)(tpt, dst, zeros)

    p0 = partials[0, :n]
    p1 = partials[1, :n]
    out = pl.pallas_call(
        _bn_body,
        out_shape=jax.ShapeDtypeStruct((n, OUT_MUL), jnp.float32),
    )(p0, p1, node_attr, bn_weight.reshape(1, OUT_MUL),
      bn_bias.reshape(1, OUT_MUL))
    return out


# speedup vs baseline: 6.1617x; 1.0243x over previous
import functools

import jax
import jax.numpy as jnp
import numpy as np
from jax import lax
from jax.experimental import pallas as pl
from jax.experimental.pallas import tpu as pltpu
from jax.experimental.pallas import tpu_sc as plsc

IN_MUL = 16
OUT_MUL = 16
NEF = 128
WNUM = IN_MUL * OUT_MUL

NC = 2
NS = 16
NW = NC * NS
CHUNK = 128
GROUP = 8
RING = GROUP * CHUNK


def _tile_split(e):
    assert e % CHUNK == 0
    total_chunks = e // CHUNK
    mx = -(-total_chunks // NW)
    assert mx % GROUP == 0
    return total_chunks, mx


def _gather_kernel(e):
    total_chunks, mx = _tile_split(e)
    per_tile = mx * CHUNK
    last = total_chunks - (NW - 1) * mx
    assert last > 0
    mesh = plsc.VectorSubcoreMesh(core_axis_name="c", subcore_axis_name="s")

    @functools.partial(
        pl.kernel,
        out_type=jax.ShapeDtypeStruct((e * IN_MUL,), jnp.float32),
        mesh=mesh,
        compiler_params=pltpu.CompilerParams(use_tc_tiling_on_sc=False, needs_layout_passes=False),
        scratch_types=[
            pltpu.VMEM((per_tile,), jnp.int32),
            pltpu.VMEM((2 * RING, IN_MUL), jnp.float32),
            pltpu.VMEM((IN_MUL * per_tile,), jnp.float32),
            pltpu.SemaphoreType.DMA,
        ],
    )
    def gather_k(table_hbm, src_hbm, out_hbm, idx_v, ring_v, t_v, sem):
        c = lax.axis_index("c")
        s = lax.axis_index("s")
        wid = s * NC + c
        base = wid * per_tile

        @pl.when(wid < NW - 1)
        def _stage_full():
            pltpu.sync_copy(src_hbm.at[pl.ds(base, per_tile)], idx_v)

        @pl.when(wid == NW - 1)
        def _stage_last():
            pltpu.sync_copy(src_hbm.at[pl.ds(base, last * CHUNK)],
                            idx_v.at[pl.ds(0, last * CHUNK)])

        iota = lax.iota(jnp.int32, 16)
        iota_128 = iota * 128
        n_groups = mx // GROUP

        def _fire(g0, half):
            for g in range(g0, g0 + GROUP):
                @pl.when(wid * mx + g < total_chunks)
                def _f(g=g):
                    pltpu.async_copy(
                        table_hbm.at[idx_v.at[pl.ds(g * CHUNK, CHUNK)]],
                        ring_v.at[pl.ds(half * RING + (g - g0) * CHUNK,
                                        CHUNK)],
                        sem,
                    )

        def _drain(g0, half):
            for g in range(g0, g0 + GROUP):
                @pl.when(wid * mx + g < total_chunks)
                def _d(g=g):
                    pltpu.make_async_copy(
                        table_hbm.at[idx_v.at[pl.ds(g * CHUNK, CHUNK)]],
                        ring_v.at[pl.ds(half * RING + (g - g0) * CHUNK,
                                        CHUNK)],
                        sem,
                    ).wait()

        _fire(0, 0)
        for gi in range(n_groups):
            g0 = gi * GROUP
            half = gi % 2
            _drain(g0, half)
            if gi + 1 < n_groups:
                _fire(g0 + GROUP, 1 - half)

            def _tbody(l, g0=g0, half=half):
                v = ring_v[half * RING + l]
                le = g0 * CHUNK + l
                off = (le >> 7) * (16 * 128) + (le & 127)
                plsc.store_scatter(t_v, [iota_128 + off], v)

            plsc.parallel_loop(0, RING, unroll=8)(_tbody)

        @pl.when(wid < NW - 1)
        def _out_full():
            pltpu.sync_copy(t_v,
                            out_hbm.at[pl.ds(base * IN_MUL,
                                             per_tile * IN_MUL)])

        @pl.when(wid == NW - 1)
        def _out_last():
            pltpu.sync_copy(t_v.at[pl.ds(0, last * CHUNK * IN_MUL)],
                            out_hbm.at[pl.ds(base * IN_MUL,
                                             last * CHUNK * IN_MUL)])

    return gather_k


def _scatter_kernel(e, n_acc):
    total_chunks, mx = _tile_split(e)
    per_tile = mx * CHUNK
    last = total_chunks - (NW - 1) * mx
    rows_per_tile = n_acc // NS
    mesh = plsc.VectorSubcoreMesh(core_axis_name="c", subcore_axis_name="s")

    @functools.partial(
        pl.kernel,
        out_type=jax.ShapeDtypeStruct((NC, n_acc, OUT_MUL), jnp.float32),
        mesh=mesh,
        compiler_params=pltpu.CompilerParams(use_tc_tiling_on_sc=False, needs_layout_passes=False),
        scratch_types=[
            pltpu.VMEM((mx, CHUNK), jnp.int32),
            pltpu.VMEM((2 * RING, OUT_MUL), jnp.float32),
            pltpu.VMEM((OUT_MUL * per_tile,), jnp.float32),
            pltpu.VMEM_SHARED((n_acc, OUT_MUL), jnp.float32),
            pltpu.SemaphoreType.DMA,
            pltpu.SemaphoreType.DMA,
        ],
    )
    def scatter_k(tp_hbm, dst_hbm, zeros_hbm, out_hbm, idx_v, ring_v, t_v,
                  acc_sh, sem, sem2):
        c = lax.axis_index("c")
        s = lax.axis_index("s")
        wid = s * NC + c
        base = wid * per_tile

        @pl.when(s == 0)
        def _init():
            pltpu.sync_copy(zeros_hbm, acc_sh)

        for g in range(mx):
            @pl.when(wid * mx + g < total_chunks)
            def _idx(g=g):
                pltpu.async_copy(
                    dst_hbm.at[pl.ds(base + g * CHUNK, CHUNK)],
                    idx_v.at[g], sem2)

        @pl.when(wid < NW - 1)
        def _stage_full():
            pltpu.async_copy(
                tp_hbm.at[pl.ds(base * OUT_MUL, per_tile * OUT_MUL)],
                t_v, sem)

        @pl.when(wid == NW - 1)
        def _stage_last():
            pltpu.async_copy(
                tp_hbm.at[pl.ds(base * OUT_MUL, last * CHUNK * OUT_MUL)],
                t_v.at[pl.ds(0, last * CHUNK * OUT_MUL)], sem)

        for g in range(mx):
            @pl.when(wid * mx + g < total_chunks)
            def _idxw(g=g):
                pltpu.make_async_copy(
                    dst_hbm.at[pl.ds(base + g * CHUNK, CHUNK)],
                    idx_v.at[g], sem2).wait()

        @pl.when(wid < NW - 1)
        def _wait_full():
            pltpu.make_async_copy(
                tp_hbm.at[pl.ds(base * OUT_MUL, per_tile * OUT_MUL)],
                t_v, sem).wait()

        @pl.when(wid == NW - 1)
        def _wait_last():
            pltpu.make_async_copy(
                tp_hbm.at[pl.ds(base * OUT_MUL, last * CHUNK * OUT_MUL)],
                t_v.at[pl.ds(0, last * CHUNK * OUT_MUL)], sem).wait()

        plsc.subcore_barrier()
        iota = lax.iota(jnp.int32, 16)
        iota_128 = iota * 128
        n_groups = mx // GROUP

        def _transpose(g0, half):
            def _tbody(l, g0=g0, half=half):
                le = g0 * CHUNK + l
                off = (le >> 7) * (16 * 128) + (le & 127)
                v = plsc.load_gather(t_v, [iota_128 + off])
                ring_v[half * RING + l] = v

            plsc.parallel_loop(0, RING, unroll=8)(_tbody)

        def _fire_adds(g0, half):
            for g in range(g0, g0 + GROUP):
                @pl.when(wid * mx + g < total_chunks)
                def _a(g=g):
                    pltpu.async_copy(
                        ring_v.at[pl.ds(half * RING + (g - g0) * CHUNK,
                                        CHUNK)],
                        acc_sh.at[idx_v.at[g]],
                        sem, add=True,
                    )

        def _drain_adds(g0, half):
            for g in range(g0, g0 + GROUP):
                @pl.when(wid * mx + g < total_chunks)
                def _w(g=g):
                    pltpu.make_async_copy(
                        ring_v.at[pl.ds(half * RING + (g - g0) * CHUNK,
                                        CHUNK)],
                        acc_sh.at[idx_v.at[g]],
                        sem,
                    ).wait()

        _transpose(0, 0)
        for gi in range(n_groups):
            g0 = gi * GROUP
            half = gi % 2
            _fire_adds(g0, half)
            if gi + 1 < n_groups:
                _transpose(g0 + GROUP, 1 - half)
            _drain_adds(g0, half)
        plsc.subcore_barrier()
        pltpu.sync_copy(
            acc_sh.at[pl.ds(s * rows_per_tile, rows_per_tile)],
            out_hbm.at[c, pl.ds(s * rows_per_tile, rows_per_tile)],
        )

    return scatter_k


def _tc_edge_body(ea_ref, x1t_ref, sh_ref, w1_ref, b1_ref, w2_ref, b2_ref,
                  r_ref, s_ref, tpt_ref):
    t = ea_ref.shape[0]
    ea = ea_ref[...].astype(jnp.bfloat16)
    h = jnp.dot(ea, w1_ref[...], preferred_element_type=jnp.float32)
    h = jnp.maximum(h + b1_ref[...], 0.0).astype(jnp.bfloat16)
    w = jnp.dot(h, w2_ref[...], preferred_element_type=jnp.float32) + b2_ref[...]
    tb = t // 128
    x1e = jnp.swapaxes(x1t_ref[...], 1, 2).reshape(t, IN_MUL)
    x1r = jnp.dot(x1e.astype(jnp.bfloat16), r_ref[...],
                  preferred_element_type=jnp.float32)
    prod = (x1r * w).astype(jnp.bfloat16)
    tp = jnp.dot(prod, s_ref[...], preferred_element_type=jnp.float32)
    tp3 = jnp.swapaxes(tp.reshape(tb, 128, OUT_MUL), 1, 2)
    tpt_ref[...] = tp3 * sh_ref[...].reshape(tb, 1, 128)


def _bn_body(p0_ref, p1_ref, na_ref, w_ref, b_ref, m_ref, out_ref):
    n = p0_ref.shape[0] * 8
    s = p0_ref[...] + p1_ref[...] + na_ref[...]
    sums = jnp.sum(s, axis=0, keepdims=True)
    mean = jnp.dot(sums, m_ref[...],
                   preferred_element_type=jnp.float32) * (1.0 / n)
    cent = s - mean
    vsum = jnp.sum(cent * cent, axis=0, keepdims=True)
    var = jnp.dot(vsum, m_ref[...],
                  preferred_element_type=jnp.float32) * (1.0 / n)
    out_ref[...] = cent * lax.rsqrt(var + 1e-5) * w_ref[...] + b_ref[...]


def kernel(node_attr, edge_index, edge_attr, edge_sh, W1, b1, W2, b2,
           bn_weight, bn_bias):
    n = node_attr.shape[0]
    e = edge_attr.shape[0]

    src = edge_index[0]
    dst = edge_index[1]

    x1t = _gather_kernel(e)(node_attr, src)

    alpha = 1.0 / np.sqrt(IN_MUL * 1)
    i_idx = np.arange(WNUM) // OUT_MUL
    k_idx = np.arange(WNUM) % OUT_MUL
    R = jnp.asarray((i_idx[None, :] == np.arange(IN_MUL)[:, None])
                    .astype(np.float32) * alpha).astype(jnp.bfloat16)
    S = jnp.asarray((k_idx[:, None] == np.arange(OUT_MUL)[None, :])
                    .astype(np.float32)).astype(jnp.bfloat16)
    T = 6400
    while e % T:
        T //= 2
    grid = (e // T,)
    ec = e // 128
    tc = T // 128
    sh3 = edge_sh.reshape(ec, 1, 128)
    x1t3 = x1t.reshape(ec, IN_MUL, 128)
    tpt3 = pl.pallas_call(
        _tc_edge_body,
        grid=grid,
        in_specs=[
            pl.BlockSpec((T, NEF), lambda i: (i, 0)),
            pl.BlockSpec((tc, IN_MUL, 128), lambda i: (i, 0, 0)),
            pl.BlockSpec((tc, 1, 128), lambda i: (i, 0, 0)),
            pl.BlockSpec((NEF, NEF), lambda i: (0, 0)),
            pl.BlockSpec((1, NEF), lambda i: (0, 0)),
            pl.BlockSpec((NEF, WNUM), lambda i: (0, 0)),
            pl.BlockSpec((1, WNUM), lambda i: (0, 0)),
            pl.BlockSpec((IN_MUL, WNUM), lambda i: (0, 0)),
            pl.BlockSpec((WNUM, OUT_MUL), lambda i: (0, 0)),
        ],
        out_specs=pl.BlockSpec((tc, OUT_MUL, 128), lambda i: (i, 0, 0)),
        out_shape=jax.ShapeDtypeStruct((ec, OUT_MUL, 128), jnp.float32),
    )(edge_attr, x1t3, sh3,
      W1.astype(jnp.bfloat16), b1.reshape(1, NEF),
      W2.astype(jnp.bfloat16), b2.reshape(1, WNUM), R, S)
    tpt = tpt3.reshape(e * OUT_MUL)

    n_acc = ((n + (NS * 8) - 1) // (NS * 8)) * (NS * 8)
    zeros = jnp.zeros((n_acc, OUT_MUL), jnp.float32)
    partials = _scatter_kernel(e, n_acc)(tpt, dst, zeros)

    assert n * OUT_MUL % 128 == 0
    nr = n * OUT_MUL // 128
    pf = partials.reshape(2 * n_acc * OUT_MUL // 128, 128)
    p0p = pf[:nr]
    p1p = pf[n_acc * OUT_MUL // 128:n_acc * OUT_MUL // 128 + nr]
    nap = node_attr.reshape(nr, 128)
    lane = np.arange(128)
    M = jnp.asarray((lane[:, None] % OUT_MUL == lane[None, :] % OUT_MUL)
                    .astype(np.float32))
    wrow = jnp.tile(bn_weight, 128 // OUT_MUL).reshape(1, 128)
    brow = jnp.tile(bn_bias, 128 // OUT_MUL).reshape(1, 128)
    outp = pl.pallas_call(
        _bn_body,
        out_shape=jax.ShapeDtypeStruct((nr, 128), jnp.float32),
    )(p0p, p1p, nap, wrow, brow, M)
    return outp.reshape(n, OUT_MUL)
